# Initial kernel scaffold; baseline (speedup 1.0000x reference)
import jax
import jax.numpy as jnp
from jax.experimental import pallas as pl


def kernel(x, edge_index_0, edge_index_1, edge_index_2, edge_index_3, edge_attr_0, edge_attr_1, edge_attr_2, edge_attr_3, params):
    raise NotImplementedError("write your pallas kernel here")


# speedup vs baseline: 1.4781x; 1.4781x over previous
import jax
import jax.numpy as jnp
from jax import lax
from jax.experimental import pallas as pl
from jax.experimental.pallas import tpu as pltpu
from jax.experimental.pallas import tpu_sc as plsc

f32 = jnp.float32
i32 = jnp.int32

NN = 10000
EE = 80000
ET = 4
ETOT = ET * EE
HH = 128

NC = 2
NS = 16
NW = NC * NS
NPW = 320
NPAD = NW * NPW
EPW = ETOT // NW
CG = 400
CP = 256
EPAD = 332800
PPW = EPAD // NW
BM = 400


def _mesh():
    return plsc.VectorSubcoreMesh(
        core_axis_name="c", subcore_axis_name="s", num_cores=NC, num_subcores=NS
    )


def _mlp2_kern(x_ref, w1_ref, b1_ref, w2_ref, b2_ref, o_ref):
    z = jnp.dot(x_ref[...], w1_ref[0], preferred_element_type=f32) + b1_ref[0]
    o_ref[...] = jnp.dot(jnp.maximum(z, 0.0), w2_ref[0],
                         preferred_element_type=f32) + b2_ref[0]


def _run_mlp2(x, w1, b1, w2, b2, nt):
    rows, kk = x.shape
    dh, do = w1.shape[2], w2.shape[2]
    nb = rows // BM
    bpt = nb // nt
    return pl.pallas_call(
        _mlp2_kern,
        grid=(nb,),
        in_specs=[
            pl.BlockSpec((BM, kk), lambda j: (j, 0)),
            pl.BlockSpec((1, kk, dh), lambda j: (j // bpt, 0, 0)),
            pl.BlockSpec((1, 1, dh), lambda j: (j // bpt, 0, 0)),
            pl.BlockSpec((1, dh, do), lambda j: (j // bpt, 0, 0)),
            pl.BlockSpec((1, 1, do), lambda j: (j // bpt, 0, 0)),
        ],
        out_specs=pl.BlockSpec((BM, do), lambda j: (j, 0)),
        out_shape=jax.ShapeDtypeStruct((rows, do), f32),
    )(x, w1, b1, w2, b2)


def _edge_msg_kern(hs_ref, hd_ref, em_ref, w1_ref, b1_ref, w2_ref, b2_ref, o_ref):
    xx = jnp.concatenate([hs_ref[...], hd_ref[...], em_ref[...]], axis=1)
    z = jnp.dot(xx, w1_ref[0], preferred_element_type=f32) + b1_ref[0]
    o_ref[...] = jnp.dot(jnp.maximum(z, 0.0), w2_ref[0],
                         preferred_element_type=f32) + b2_ref[0]


def _edge_msg(hs, hd, em, w1, b1, w2, b2):
    nb = ETOT // BM
    bpt = nb // ET
    eb = lambda j: (j, 0)
    wb3 = lambda j: (j // bpt, 0, 0)
    return pl.pallas_call(
        _edge_msg_kern,
        grid=(nb,),
        in_specs=[
            pl.BlockSpec((BM, HH), eb),
            pl.BlockSpec((BM, HH), eb),
            pl.BlockSpec((BM, HH), eb),
            pl.BlockSpec((1, 3 * HH, 2 * HH), wb3),
            pl.BlockSpec((1, 1, 2 * HH), wb3),
            pl.BlockSpec((1, 2 * HH, HH), wb3),
            pl.BlockSpec((1, 1, HH), wb3),
        ],
        out_specs=pl.BlockSpec((BM, HH), eb),
        out_shape=jax.ShapeDtypeStruct((ETOT, HH), f32),
    )(hs, hd, em, w1, b1, w2, b2)


def _ln(t, g, b):
    m = jnp.mean(t, axis=1, keepdims=True)
    v = jnp.mean((t - m) ** 2, axis=1, keepdims=True)
    return (t - m) * lax.rsqrt(v + 1e-5) * g + b


def _fuse_kern(h_ref, a_ref, v1_ref, c1_ref, v2_ref, c2_ref, g_ref, b_ref,
               o_ref):
    h = h_ref[...]
    a = a_ref[...]
    a = jnp.where(a == -jnp.inf, 0.0, a)
    xx = jnp.concatenate([h, a], axis=1)
    u = jnp.maximum(jnp.dot(xx, v1_ref[...], preferred_element_type=f32)
                    + c1_ref[...], 0.0)
    u = jnp.dot(u, v2_ref[...], preferred_element_type=f32) + c2_ref[...]
    o_ref[...] = _ln(h + u, g_ref[...], b_ref[...])


def _fuse(h, agg, v1, c1, v2, c2, g, b):
    nb = NN // BM
    return pl.pallas_call(
        _fuse_kern,
        grid=(nb,),
        in_specs=[
            pl.BlockSpec((BM, HH), lambda i: (i, 0)),
            pl.BlockSpec((BM, HH), lambda i: (i, 0)),
            pl.BlockSpec((2 * HH, 2 * HH), lambda i: (0, 0)),
            pl.BlockSpec((1, 2 * HH), lambda i: (0, 0)),
            pl.BlockSpec((2 * HH, HH), lambda i: (0, 0)),
            pl.BlockSpec((1, HH), lambda i: (0, 0)),
            pl.BlockSpec((1, HH), lambda i: (0, 0)),
            pl.BlockSpec((1, HH), lambda i: (0, 0)),
        ],
        out_specs=pl.BlockSpec((BM, HH), lambda i: (i, 0)),
        out_shape=jax.ShapeDtypeStruct((NN, HH), f32),
    )(h, agg, v1, c1, v2, c2, g, b)


def _gate_kern(x_ref, gw_ref, gb_ref, o_ref):
    z = jnp.sum(x_ref[...] * gw_ref[...], axis=1, keepdims=True) + gb_ref[...]
    o_ref[...] = 1.0 / (1.0 + jnp.exp(-z))


def _gate(ea_s, gw, gb):
    bg = 1600
    nb = ETOT // bg
    return pl.pallas_call(
        _gate_kern,
        grid=(nb,),
        in_specs=[
            pl.BlockSpec((bg, 16), lambda j: (j, 0)),
            pl.BlockSpec((1, 16), lambda j: (0, 0)),
            pl.BlockSpec((1, 1), lambda j: (0, 0)),
        ],
        out_specs=pl.BlockSpec((bg, 1), lambda j: (j, 0)),
        out_shape=jax.ShapeDtypeStruct((ETOT, 1), f32),
    )(ea_s, gw, gb)


def _head_kern(h_ref, r1_ref, rb1_ref, r2_ref, rb2_ref, np_ref, gm_ref):
    i = pl.program_id(0)
    h = h_ref[...]
    z = jnp.maximum(jnp.dot(h, r1_ref[...], preferred_element_type=f32)
                    + rb1_ref[...], 0.0)
    npred = jnp.dot(z, r2_ref[...], preferred_element_type=f32) + rb2_ref[...]
    np_ref[...] = npred[:, :1]
    bmx = jnp.max(h, axis=0, keepdims=True)

    @pl.when(i == 0)
    def _():
        gm_ref[...] = bmx

    @pl.when(i != 0)
    def _():
        gm_ref[...] = jnp.maximum(gm_ref[...], bmx)


def _head(h, r1, rb1, r2, rb2):
    nb = NN // BM
    return pl.pallas_call(
        _head_kern,
        grid=(nb,),
        in_specs=[
            pl.BlockSpec((BM, HH), lambda i: (i, 0)),
            pl.BlockSpec((HH, HH // 2), lambda i: (0, 0)),
            pl.BlockSpec((1, HH // 2), lambda i: (0, 0)),
            pl.BlockSpec((HH // 2, 8), lambda i: (0, 0)),
            pl.BlockSpec((1, 8), lambda i: (0, 0)),
        ],
        out_specs=[
            pl.BlockSpec((BM, 1), lambda i: (i, 0)),
            pl.BlockSpec((1, HH), lambda i: (0, 0)),
        ],
        out_shape=[
            jax.ShapeDtypeStruct((NN, 1), f32),
            jax.ShapeDtypeStruct((1, HH), f32),
        ],
    )(h, r1, rb1, r2, rb2)


def _ghead_kern(g_ref, g1_ref, gb1_ref, g2_ref, gb2_ref, o_ref):
    z = jnp.maximum(jnp.dot(g_ref[...], g1_ref[...], preferred_element_type=f32)
                    + gb1_ref[...], 0.0)
    o_ref[...] = jnp.dot(z, g2_ref[...], preferred_element_type=f32) + gb2_ref[...]


def _ghead(gm, g1, gb1, g2, gb2):
    return pl.pallas_call(
        _ghead_kern,
        out_shape=jax.ShapeDtypeStruct((1, 8), f32),
    )(gm, g1, gb1, g2, gb2)


def _sc_gather2_body(h_hbm, src_hbm, dst_hbm, hs_hbm, hd_hbm, idx_v, rows_v,
                     sem):
    wid = lax.axis_index("c") * NS + lax.axis_index("s")
    base = wid * EPW

    def chunk(ci, _):
        off = base + ci * CG
        pltpu.sync_copy(src_hbm.at[pl.ds(off, CG)], idx_v)
        pltpu.async_copy(h_hbm.at[idx_v], rows_v, sem).wait()
        pltpu.sync_copy(rows_v, hs_hbm.at[pl.ds(off, CG)])
        pltpu.sync_copy(dst_hbm.at[pl.ds(off, CG)], idx_v)
        pltpu.async_copy(h_hbm.at[idx_v], rows_v, sem).wait()
        pltpu.sync_copy(rows_v, hd_hbm.at[pl.ds(off, CG)])
        return 0

    lax.fori_loop(0, EPW // CG, chunk, 0)


def _sc_gather2(h, src, dst):
    return pl.kernel(
        _sc_gather2_body,
        out_type=(
            jax.ShapeDtypeStruct((ETOT, HH), f32),
            jax.ShapeDtypeStruct((ETOT, HH), f32),
        ),
        mesh=_mesh(),
        scratch_types=[
            pltpu.VMEM((CG,), i32),
            pltpu.VMEM((CG, HH), f32),
            pltpu.SemaphoreType.DMA,
        ],
    )(h, src, dst)


def _sc_segsum_body(m_hbm, idx_hbm, dst_hbm, bnd_hbm, out_hbm,
                    bnd_v, idx_v, dst_v, rows_v, acc_v, sem):
    wid = lax.axis_index("c") * NS + lax.axis_index("s")
    n0 = wid * NPW
    pltpu.sync_copy(bnd_hbm, bnd_v)
    bv = bnd_v[pl.ds(wid, 16)]
    e0 = bv[0]
    e1 = bv[1]
    start = (e0 // 8) * 8
    nch = (e1 - start + CP - 1) // CP

    zero = jnp.zeros((16,), f32)

    def initrow(r, _):
        for f in range(HH // 16):
            acc_v[r, pl.ds(f * 16, 16)] = zero
        return 0

    lax.fori_loop(0, NPW, initrow, 0)

    def chunk(ci, _):
        off = start + ci * CP
        pltpu.sync_copy(idx_hbm.at[pl.ds(off, CP)], idx_v)
        pltpu.sync_copy(dst_hbm.at[pl.ds(off, CP)], dst_v.at[pl.ds(0, CP)])
        pltpu.async_copy(m_hbm.at[idx_v], rows_v, sem).wait()

        def edge(e, _):
            row = dst_v[pl.ds(e, 16)][0] - n0
            ok = (row >= 0) & (off + e < e1)
            rowc = jnp.where(ok, row, 0)
            for f in range(HH // 16):
                sl = pl.ds(f * 16, 16)
                a = acc_v[rowc, sl]
                v = rows_v[e, sl]
                acc_v[rowc, sl] = jnp.where(ok, a + v, a)
            return 0

        lax.fori_loop(0, CP, edge, 0)
        return 0

    lax.fori_loop(0, nch, chunk, 0)
    pltpu.sync_copy(acc_v, out_hbm.at[pl.ds(n0, NPW)])


def _sc_segsum(m, perm, dsts, bounds):
    return pl.kernel(
        _sc_segsum_body,
        out_type=jax.ShapeDtypeStruct((NPAD, HH), f32),
        mesh=_mesh(),
        scratch_types=[
            pltpu.VMEM((48,), i32),
            pltpu.VMEM((CP,), i32),
            pltpu.VMEM((CP + 16,), i32),
            pltpu.VMEM((CP, HH), f32),
            pltpu.VMEM((NPW, HH), f32),
            pltpu.SemaphoreType.DMA,
        ],
    )(m, perm, dsts, bounds)


def _sc_prop_body(h_hbm, src_hbm, w_hbm, dst_hbm, bnd_hbm, out_hbm,
                  bnd_v, idx_v, w_v, dst_v, rows_v, acc_v, sem):
    wid = lax.axis_index("c") * NS + lax.axis_index("s")
    n0 = wid * NPW
    pltpu.sync_copy(bnd_hbm, bnd_v)
    bv = bnd_v[pl.ds(wid, 16)]
    e0 = bv[0]
    e1 = bv[1]
    start = (e0 // 8) * 8
    nch = (e1 - start + CP - 1) // CP

    neg = jnp.full((16,), -jnp.inf, f32)

    def initrow(r, _):
        for f in range(HH // 16):
            acc_v[r, pl.ds(f * 16, 16)] = neg
        return 0

    lax.fori_loop(0, NPW, initrow, 0)

    def chunk(ci, _):
        off = start + ci * CP
        pltpu.sync_copy(src_hbm.at[pl.ds(off, CP)], idx_v)
        pltpu.sync_copy(dst_hbm.at[pl.ds(off, CP)], dst_v.at[pl.ds(0, CP)])
        pltpu.sync_copy(w_hbm.at[pl.ds(off, CP)], w_v.at[pl.ds(0, CP)])
        pltpu.async_copy(h_hbm.at[idx_v], rows_v, sem).wait()

        def edge(e, _):
            row = dst_v[pl.ds(e, 16)][0] - n0
            ok = (row >= 0) & (off + e < e1)
            rowc = jnp.where(ok, row, 0)
            wgt = w_v[pl.ds(e, 16)][0]
            for f in range(HH // 16):
                sl = pl.ds(f * 16, 16)
                a = acc_v[rowc, sl]
                v = rows_v[e, sl] * wgt
                acc_v[rowc, sl] = jnp.where(ok, jnp.maximum(a, v), a)
            return 0

        lax.fori_loop(0, CP, edge, 0)
        return 0

    lax.fori_loop(0, nch, chunk, 0)
    pltpu.sync_copy(acc_v, out_hbm.at[pl.ds(n0, NPW)])


def _sc_prop(h, srcs, ws, dsts, bounds):
    return pl.kernel(
        _sc_prop_body,
        out_type=jax.ShapeDtypeStruct((NPAD, HH), f32),
        mesh=_mesh(),
        scratch_types=[
            pltpu.VMEM((48,), i32),
            pltpu.VMEM((CP,), i32),
            pltpu.VMEM((CP + 16,), f32),
            pltpu.VMEM((CP + 16,), i32),
            pltpu.VMEM((CP, HH), f32),
            pltpu.VMEM((NPW, HH), f32),
            pltpu.SemaphoreType.DMA,
        ],
    )(h, srcs, ws, dsts, bounds)


def kernel(x, edge_index_0, edge_index_1, edge_index_2, edge_index_3,
           edge_attr_0, edge_attr_1, edge_attr_2, edge_attr_3, params):
    p = params
    eidxs = (edge_index_0, edge_index_1, edge_index_2, edge_index_3)
    eattrs = (edge_attr_0, edge_attr_1, edge_attr_2, edge_attr_3)

    src_all = jnp.concatenate([e[0] for e in eidxs])
    dst_all = jnp.concatenate([e[1] for e in eidxs])
    ea_all = jnp.concatenate(eattrs, axis=0)
    ea_pad = jnp.pad(ea_all, ((0, 0), (0, 9)))

    perm = jnp.argsort(dst_all).astype(i32)
    dsts_sorted = dst_all[perm]
    srcs_s = jnp.pad(src_all[perm], (0, EPAD - ETOT))
    dsts_s = jnp.pad(dsts_sorted, (0, EPAD - ETOT))
    perm_p = jnp.pad(perm, (0, EPAD - ETOT))
    qnodes = jnp.arange(0, NPAD + 1, NPW, dtype=i32)
    bounds = jnp.searchsorted(dsts_sorted, qnodes, side="left").astype(i32)
    bounds = jnp.pad(bounds, (0, 48 - bounds.shape[0]))

    ne = p["node_enc"]
    x16 = jnp.pad(x, ((0, 0), (0, 2)))
    w1n = jnp.pad(ne["l1"]["w"], ((0, 2), (0, 0)))[None]
    h = _run_mlp2(x16, w1n, ne["l1"]["b"][None, None], ne["l2"]["w"][None],
                  ne["l2"]["b"][None, None], nt=1)

    ee = p["edge_encs"]
    w1e = jnp.stack([jnp.pad(ee[k]["l1"]["w"], ((0, 1), (0, 0)))
                     for k in range(ET)])
    b1e = jnp.stack([ee[k]["l1"]["b"] for k in range(ET)])[:, None, :]
    w2e = jnp.stack([ee[k]["l2"]["w"] for k in range(ET)])
    b2e = jnp.stack([ee[k]["l2"]["b"] for k in range(ET)])[:, None, :]
    emb = _run_mlp2(ea_pad[:, :8], w1e, b1e, w2e, b2e, nt=ET)

    gw = jnp.pad(p["delay_gate"]["w"], ((0, 9), (0, 0))).T
    gb = p["delay_gate"]["b"][None]
    w_all = _gate(ea_pad, gw, gb).reshape(ETOT)
    w_s = jnp.pad(w_all[perm], (0, EPAD - ETOT))

    for lp in p["layers"]:
        w1l = jnp.stack([lp["edge_mlps"][k]["l1"]["w"] for k in range(ET)])
        b1l = jnp.stack([lp["edge_mlps"][k]["l1"]["b"] for k in range(ET)])[:, None, :]
        w2l = jnp.stack([lp["edge_mlps"][k]["l2"]["w"] for k in range(ET)])
        b2l = jnp.stack([lp["edge_mlps"][k]["l2"]["b"] for k in range(ET)])[:, None, :]
        hs, hd = _sc_gather2(h, src_all, dst_all)
        m = _edge_msg(hs, hd, emb, w1l, b1l, w2l, b2l)
        agg = _sc_segsum(m, perm_p, dsts_s, bounds)
        nm = lp["node_mlp"]
        h = _fuse(h, agg, nm["l1"]["w"], nm["l1"]["b"][None],
                  nm["l2"]["w"], nm["l2"]["b"][None],
                  lp["ln_g"][None], lp["ln_b"][None])

    for fp in p["prop_fuses"]:
        agg = _sc_prop(h, srcs_s, w_s, dsts_s, bounds)
        mm = fp["mlp"]
        h = _fuse(h, agg, mm["l1"]["w"], mm["l1"]["b"][None],
                  mm["l2"]["w"], mm["l2"]["b"][None],
                  fp["ln_g"][None], fp["ln_b"][None])

    rh = p["reg_head"]
    r2 = jnp.pad(rh["l2"]["w"], ((0, 0), (0, 7)))
    rb2 = jnp.pad(rh["l2"]["b"], (0, 7))[None]
    npred2, gmax = _head(h, rh["l1"]["w"], rh["l1"]["b"][None], r2, rb2)
    node_pred = npred2[:, 0]

    gh = p["graph_head"]
    g2 = jnp.pad(gh["l2"]["w"], ((0, 0), (0, 7)))
    gb2 = jnp.pad(gh["l2"]["b"], (0, 7))[None]
    gp = _ghead(gmax, gh["l1"]["w"], gh["l1"]["b"][None], g2, gb2)
    graph_pred = gp[0, :1]

    return (node_pred, graph_pred)
